# Initial kernel scaffold; baseline (speedup 1.0000x reference)
import jax
import jax.numpy as jnp
from jax.experimental import pallas as pl


def kernel(x, w_map, b_map):
    raise NotImplementedError("write your pallas kernel here")


# speedup vs baseline: 1.2699x; 1.2699x over previous
import math
from functools import partial

import numpy as np
import jax
import jax.numpy as jnp
from jax.experimental import pallas as pl
from jax.experimental.pallas import tpu as pltpu


def _bilinear_matrix(in_size: int, out_size: int) -> np.ndarray:
    scale = in_size / out_size
    src = (np.arange(out_size, dtype=np.float64) + 0.5) * scale - 0.5
    src = np.maximum(src, 0.0)
    i0 = np.minimum(np.floor(src).astype(np.int64), in_size - 1)
    i1 = np.minimum(i0 + 1, in_size - 1)
    frac = src - i0
    m = np.zeros((out_size, in_size), dtype=np.float64)
    rows = np.arange(out_size)
    m[rows, i0] += 1.0 - frac
    m[rows, i1] += frac
    return m


def _staged_bilinear(size: int, multiplier: float, n_stages: int) -> np.ndarray:
    m = np.eye(size, dtype=np.float64)
    cur = size
    for _ in range(n_stages):
        nxt = int(math.floor(cur * multiplier))
        m = _bilinear_matrix(cur, nxt) @ m
        cur = nxt
    return m


def _rescale_body(x_ref, awt_ref, ah_ref, w_ref, b_ref, o_ref, *, C, H, Ho, Cout):
    x = x_ref[0]
    y = jnp.dot(x, awt_ref[...], preferred_element_type=jnp.float32)
    ah = ah_ref[...]
    z = [jnp.dot(ah, y[c * H:(c + 1) * H, :], preferred_element_type=jnp.float32)
         for c in range(C)]
    for co in range(Cout):
        acc = z[0] * w_ref[co, 0]
        for c in range(1, C):
            acc = acc + z[c] * w_ref[co, c]
        o_ref[0, co * Ho:(co + 1) * Ho, :] = acc + b_ref[co]


def kernel(x, w_map, b_map):
    N, C, H, W = x.shape
    Cout = int(w_map.shape[0])
    a_h = _staged_bilinear(H, 0.5, 1)
    a_w = _staged_bilinear(W, 0.5, 1)
    Ho, Wo = a_h.shape[0], a_w.shape[0]

    awt = jnp.asarray(a_w.T.astype(np.float32))
    ah = jnp.asarray(a_h.astype(np.float32))

    x_in = x.reshape(N, C * H, W)
    out = pl.pallas_call(
        partial(_rescale_body, C=C, H=H, Ho=Ho, Cout=Cout),
        out_shape=jax.ShapeDtypeStruct((N, Cout * Ho, Wo), x.dtype),
        grid=(N,),
        in_specs=[
            pl.BlockSpec((1, C * H, W), lambda n: (n, 0, 0)),
            pl.BlockSpec((W, Wo), lambda n: (0, 0)),
            pl.BlockSpec((Ho, H), lambda n: (0, 0)),
            pl.BlockSpec(memory_space=pltpu.SMEM),
            pl.BlockSpec(memory_space=pltpu.SMEM),
        ],
        out_specs=pl.BlockSpec((1, Cout * Ho, Wo), lambda n: (n, 0, 0)),
        compiler_params=pltpu.CompilerParams(
            dimension_semantics=("parallel",),
        ),
    )(x_in, awt, ah, jnp.asarray(w_map, jnp.float32), jnp.asarray(b_map, jnp.float32))
    return out.reshape(N, Cout, Ho, Wo)


# speedup vs baseline: 1.7266x; 1.3596x over previous
import math
from functools import partial

import numpy as np
import jax
import jax.numpy as jnp
from jax.experimental import pallas as pl
from jax.experimental.pallas import tpu as pltpu


def _bilinear_matrix(in_size: int, out_size: int) -> np.ndarray:
    scale = in_size / out_size
    src = (np.arange(out_size, dtype=np.float64) + 0.5) * scale - 0.5
    src = np.maximum(src, 0.0)
    i0 = np.minimum(np.floor(src).astype(np.int64), in_size - 1)
    i1 = np.minimum(i0 + 1, in_size - 1)
    frac = src - i0
    m = np.zeros((out_size, in_size), dtype=np.float64)
    rows = np.arange(out_size)
    m[rows, i0] += 1.0 - frac
    m[rows, i1] += frac
    return m


def _staged_bilinear(size: int, multiplier: float, n_stages: int) -> np.ndarray:
    m = np.eye(size, dtype=np.float64)
    cur = size
    for _ in range(n_stages):
        nxt = int(math.floor(cur * multiplier))
        m = _bilinear_matrix(cur, nxt) @ m
        cur = nxt
    return m


def _rescale_body(x_ref, awt_ref, ah_ref, w_ref, b_ref, o_ref, *, BB, C, H, Ho, Cout):
    x = x_ref[...].reshape(BB * C * H, x_ref.shape[2])
    y = jnp.dot(x, awt_ref[...], preferred_element_type=jnp.float32)
    ah = ah_ref[...]
    for b in range(BB):
        z = [jnp.dot(ah, y[(b * C + c) * H:(b * C + c + 1) * H, :],
                     preferred_element_type=jnp.float32)
             for c in range(C)]
        for co in range(Cout):
            acc = z[0] * w_ref[co, 0]
            for c in range(1, C):
                acc = acc + z[c] * w_ref[co, c]
            o_ref[b, co * Ho:(co + 1) * Ho, :] = acc + b_ref[co]


def kernel(x, w_map, b_map):
    N, C, H, W = x.shape
    Cout = int(w_map.shape[0])
    a_h = _staged_bilinear(H, 0.5, 1)
    a_w = _staged_bilinear(W, 0.5, 1)
    Ho, Wo = a_h.shape[0], a_w.shape[0]

    awt = jnp.asarray(a_w.T.astype(np.float32))
    ah = jnp.asarray(a_h.astype(np.float32))

    BB = 2 if N % 2 == 0 else 1
    x_in = x.reshape(N, C * H, W)
    out = pl.pallas_call(
        partial(_rescale_body, BB=BB, C=C, H=H, Ho=Ho, Cout=Cout),
        out_shape=jax.ShapeDtypeStruct((N, Cout * Ho, Wo), x.dtype),
        grid=(N // BB,),
        in_specs=[
            pl.BlockSpec((BB, C * H, W), lambda n: (n, 0, 0)),
            pl.BlockSpec((W, Wo), lambda n: (0, 0)),
            pl.BlockSpec((Ho, H), lambda n: (0, 0)),
            pl.BlockSpec(memory_space=pltpu.SMEM),
            pl.BlockSpec(memory_space=pltpu.SMEM),
        ],
        out_specs=pl.BlockSpec((BB, Cout * Ho, Wo), lambda n: (n, 0, 0)),
        compiler_params=pltpu.CompilerParams(
            dimension_semantics=("parallel",),
        ),
    )(x_in, awt, ah, jnp.asarray(w_map, jnp.float32), jnp.asarray(b_map, jnp.float32))
    return out.reshape(N, Cout, Ho, Wo)


# speedup vs baseline: 2.1151x; 1.2250x over previous
import math
from functools import partial

import numpy as np
import jax
import jax.numpy as jnp
from jax.experimental import pallas as pl
from jax.experimental.pallas import tpu as pltpu


def _bilinear_matrix(in_size: int, out_size: int) -> np.ndarray:
    scale = in_size / out_size
    src = (np.arange(out_size, dtype=np.float64) + 0.5) * scale - 0.5
    src = np.maximum(src, 0.0)
    i0 = np.minimum(np.floor(src).astype(np.int64), in_size - 1)
    i1 = np.minimum(i0 + 1, in_size - 1)
    frac = src - i0
    m = np.zeros((out_size, in_size), dtype=np.float64)
    rows = np.arange(out_size)
    m[rows, i0] += 1.0 - frac
    m[rows, i1] += frac
    return m


def _staged_bilinear(size: int, multiplier: float, n_stages: int) -> np.ndarray:
    m = np.eye(size, dtype=np.float64)
    cur = size
    for _ in range(n_stages):
        nxt = int(math.floor(cur * multiplier))
        m = _bilinear_matrix(cur, nxt) @ m
        cur = nxt
    return m


def _rescale_body(x_ref, awt_ref, ah_ref, w_ref, b_ref, o_ref, *, BB, C, H, Ho, Cout):
    x = x_ref[...].reshape(BB * C * H, x_ref.shape[2])
    y = jnp.dot(x, awt_ref[...], preferred_element_type=jnp.float32)
    ah = ah_ref[...]
    for b in range(BB):
        z = [jnp.dot(ah, y[(b * C + c) * H:(b * C + c + 1) * H, :],
                     preferred_element_type=jnp.float32)
             for c in range(C)]
        for co in range(Cout):
            acc = z[0] * w_ref[co, 0]
            for c in range(1, C):
                acc = acc + z[c] * w_ref[co, c]
            o_ref[b, co * Ho:(co + 1) * Ho, :] = acc + b_ref[co]


def kernel(x, w_map, b_map):
    N, C, H, W = x.shape
    Cout = int(w_map.shape[0])
    a_h = _staged_bilinear(H, 0.5, 1)
    a_w = _staged_bilinear(W, 0.5, 1)
    Ho, Wo = a_h.shape[0], a_w.shape[0]

    awt = jnp.asarray(a_w.T.astype(np.float32))
    ah = jnp.asarray(a_h.astype(np.float32))

    BB = 4 if N % 4 == 0 else (2 if N % 2 == 0 else 1)
    x_in = x.reshape(N, C * H, W)
    out = pl.pallas_call(
        partial(_rescale_body, BB=BB, C=C, H=H, Ho=Ho, Cout=Cout),
        out_shape=jax.ShapeDtypeStruct((N, Cout * Ho, Wo), x.dtype),
        grid=(N // BB,),
        in_specs=[
            pl.BlockSpec((BB, C * H, W), lambda n: (n, 0, 0)),
            pl.BlockSpec((W, Wo), lambda n: (0, 0)),
            pl.BlockSpec((Ho, H), lambda n: (0, 0)),
            pl.BlockSpec(memory_space=pltpu.SMEM),
            pl.BlockSpec(memory_space=pltpu.SMEM),
        ],
        out_specs=pl.BlockSpec((BB, Cout * Ho, Wo), lambda n: (n, 0, 0)),
        compiler_params=pltpu.CompilerParams(
            dimension_semantics=("parallel",),
        ),
    )(x_in, awt, ah, jnp.asarray(w_map, jnp.float32), jnp.asarray(b_map, jnp.float32))
    return out.reshape(N, Cout, Ho, Wo)


# speedup vs baseline: 2.2362x; 1.0572x over previous
import math
from functools import partial

import numpy as np
import jax
import jax.numpy as jnp
from jax.experimental import pallas as pl
from jax.experimental.pallas import tpu as pltpu


def _bilinear_matrix(in_size: int, out_size: int) -> np.ndarray:
    scale = in_size / out_size
    src = (np.arange(out_size, dtype=np.float64) + 0.5) * scale - 0.5
    src = np.maximum(src, 0.0)
    i0 = np.minimum(np.floor(src).astype(np.int64), in_size - 1)
    i1 = np.minimum(i0 + 1, in_size - 1)
    frac = src - i0
    m = np.zeros((out_size, in_size), dtype=np.float64)
    rows = np.arange(out_size)
    m[rows, i0] += 1.0 - frac
    m[rows, i1] += frac
    return m


def _staged_bilinear(size: int, multiplier: float, n_stages: int) -> np.ndarray:
    m = np.eye(size, dtype=np.float64)
    cur = size
    for _ in range(n_stages):
        nxt = int(math.floor(cur * multiplier))
        m = _bilinear_matrix(cur, nxt) @ m
        cur = nxt
    return m


def _rescale_body(x_ref, awt_ref, ah_ref, w_ref, b_ref, o_ref, *, BB, C, H, Ho, Cout):
    x = x_ref[...].reshape(BB * C * H, x_ref.shape[2])
    y = jnp.dot(x, awt_ref[...], preferred_element_type=jnp.float32)
    ah = ah_ref[...]
    for b in range(BB):
        z = [jnp.dot(ah, y[(b * C + c) * H:(b * C + c + 1) * H, :],
                     preferred_element_type=jnp.float32)
             for c in range(C)]
        for co in range(Cout):
            acc = z[0] * w_ref[co, 0]
            for c in range(1, C):
                acc = acc + z[c] * w_ref[co, c]
            o_ref[b, co * Ho:(co + 1) * Ho, :] = acc + b_ref[co]


def kernel(x, w_map, b_map):
    N, C, H, W = x.shape
    Cout = int(w_map.shape[0])
    a_h = _staged_bilinear(H, 0.5, 1)
    a_w = _staged_bilinear(W, 0.5, 1)
    Ho, Wo = a_h.shape[0], a_w.shape[0]

    awt = jnp.asarray(a_w.T.astype(np.float32))
    ah = jnp.asarray(a_h.astype(np.float32))

    BB = next((b for b in (8, 4, 2) if N % b == 0), 1)
    x_in = x.reshape(N, C * H, W)
    out = pl.pallas_call(
        partial(_rescale_body, BB=BB, C=C, H=H, Ho=Ho, Cout=Cout),
        out_shape=jax.ShapeDtypeStruct((N, Cout * Ho, Wo), x.dtype),
        grid=(N // BB,),
        in_specs=[
            pl.BlockSpec((BB, C * H, W), lambda n: (n, 0, 0)),
            pl.BlockSpec((W, Wo), lambda n: (0, 0)),
            pl.BlockSpec((Ho, H), lambda n: (0, 0)),
            pl.BlockSpec(memory_space=pltpu.SMEM),
            pl.BlockSpec(memory_space=pltpu.SMEM),
        ],
        out_specs=pl.BlockSpec((BB, Cout * Ho, Wo), lambda n: (n, 0, 0)),
        compiler_params=pltpu.CompilerParams(
            dimension_semantics=("parallel",),
        ),
    )(x_in, awt, ah, jnp.asarray(w_map, jnp.float32), jnp.asarray(b_map, jnp.float32))
    return out.reshape(N, Cout, Ho, Wo)
